# baseline (speedup 1.0000x reference)
import functools

import jax
import jax.numpy as jnp
from jax import lax
from jax.experimental import pallas as pl
from jax.experimental.pallas import tpu as pltpu
from jax.experimental.pallas import tpu_sc as plsc

_HID = 768
_MAX_2D = 1024
_PAD = 1
_EPS = 1e-5
_B, _S = 32, 512
_T = 32
_NCHUNK = _S // _T
_L = 16
_NSL = _HID // _L
_SEG = 128 // _L
_PW = 40
_POSPAD = 528


def _sc_body(ids_hbm, bbT_hbm, word_hbm, pos_hbm, xyc_hbm, hwc_hbm,
             gamma_hbm, beta_hbm, out_hbm,
             shxy, ids_v, bb_v, rm_v, gamma_v, beta_v, ig1c, ig2c,
             pos_lin, w_a, w_b, g1_v, g2_v,
             sem_wa, sem_wb, sem_p, sem_s, osem_a, osem_b):
    sid = lax.axis_index("s")
    b = sid * 2 + lax.axis_index("c")

    pltpu.sync_copy(ids_hbm.at[b], ids_v)
    for c in range(4):
        pltpu.sync_copy(bbT_hbm.at[c, b], bb_v.at[c])
    pltpu.sync_copy(gamma_hbm, gamma_v)
    pltpu.sync_copy(beta_hbm, beta_v)

    ssl = pl.ds(sid * 128, 128)
    pltpu.sync_copy(xyc_hbm.at[ssl], shxy.at[ssl])
    plsc.subcore_barrier()

    def idx_body(i, carry):
        sl = pl.ds(i * _L, _L)
        ids = ids_v[sl]
        m = jnp.where(ids == _PAD, 0, 1).astype(jnp.int32)
        cs = lax.cumsum(m, axis=0) + carry
        rm_v[sl] = ((cs + 1) << 1) | m
        return carry + jnp.sum(m)

    lax.fori_loop(0, _S // _L, idx_body, jnp.int32(0))

    def chunk_scalars(k):
        c0 = k * _T
        r0 = rm_v[pl.ds(c0, _L)]
        r1 = rm_v[pl.ds(c0 + _L, _L)]
        maxr = jnp.maximum(jnp.max(r0), jnp.max(r1)) >> 1
        n = jnp.sum(r0 & 1) + jnp.sum(r1 & 1)
        cbase = maxr - n - 1
        base8 = pl.multiple_of((cbase + 1) & ~7, 8)
        return c0, base8

    def build_spatial_idx(c0):
        for half in range(2):
            sl16 = pl.ds(c0 + half * _L, _L)
            b0 = bb_v[0, sl16]
            b1 = bb_v[1, sl16]
            b2 = bb_v[2, sl16]
            b3 = bb_v[3, sl16]
            o = half * _L
            ig1c[pl.ds(o, _L)] = b0
            ig1c[pl.ds(32 + o, _L)] = b1 + _MAX_2D
            ig1c[pl.ds(64 + o, _L)] = b2
            ig1c[pl.ds(96 + o, _L)] = b3 + _MAX_2D
            ig2c[pl.ds(o, _L)] = jnp.clip(b3 - b1, 0, _MAX_2D - 1)
            ig2c[pl.ds(32 + o, _L)] = jnp.clip(b2 - b0, 0, _MAX_2D - 1) + _MAX_2D

    def issue_side(base8):
        pltpu.async_copy(pos_hbm.at[pl.ds(base8, _PW)], pos_lin, sem_p)
        pltpu.async_copy(xyc_hbm.at[ig1c], g1_v, sem_s)
        pltpu.async_copy(hwc_hbm.at[ig2c], g2_v, sem_s)

    def wait_side():
        pltpu.make_async_copy(pos_hbm.at[pl.ds(0, _PW)], pos_lin, sem_p).wait()
        pltpu.make_async_copy(xyc_hbm.at[pl.ds(0, 128)], g1_v, sem_s).wait()
        pltpu.make_async_copy(hwc_hbm.at[pl.ds(0, 64)], g2_v, sem_s).wait()

    def issue_word(c0, wbuf, sem):
        pltpu.async_copy(word_hbm.at[ids_v.at[pl.ds(c0, _T)]], wbuf, sem)

    def wait_word(wbuf, sem):
        pltpu.make_async_copy(word_hbm.at[pl.ds(0, _T)], wbuf, sem).wait()

    def out_copy(c0, wbuf, osem):
        pltpu.async_copy(wbuf, out_hbm.at[b, pl.ds(c0, _T)], osem)

    def wait_out(wbuf, osem):
        pltpu.make_async_copy(wbuf, out_hbm.at[b, pl.ds(0, _T)], osem).wait()

    def compute(c0, base8, wbuf):
        @plsc.parallel_loop(0, _T, 1, unroll=2)
        def token_body(t):
            rm = rm_v[pl.ds(c0 + t, _L)][0]
            q = (rm >> 1) - base8
            mt = (rm & 1).astype(jnp.float32)
            vs = [jnp.zeros((_L,), jnp.float32) for _ in range(4)]
            vq = [jnp.zeros((_L,), jnp.float32) for _ in range(4)]
            for s in range(_NSL):
                dsl = pl.ds(s * _L, _L)
                seg = s // _SEG
                sub = pl.ds((s % _SEG) * _L + (seg - 4) * 0, _L)
                v = wbuf[t, dsl] + pos_lin[q, dsl] * mt
                if seg < 4:
                    v = v + g1_v[seg * _T + t, sub]
                else:
                    v = v + g2_v[(seg - 4) * _T + t, sub]
                wbuf[t, dsl] = v
                a = s % 4
                vs[a] = vs[a] + v
                vq[a] = vq[a] + v * v
            vsum = (vs[0] + vs[1]) + (vs[2] + vs[3])
            vsq = (vq[0] + vq[1]) + (vq[2] + vq[3])
            mean = jnp.sum(vsum) * (1.0 / _HID)
            var = jnp.sum(vsq) * (1.0 / _HID) - mean * mean + _EPS
            xv = jnp.full((_L,), var, dtype=jnp.float32)
            iv = lax.bitcast_convert_type(
                jnp.int32(0x5F3759DF)
                - (lax.bitcast_convert_type(xv, jnp.int32) >> 1),
                jnp.float32)
            for _ in range(3):
                iv = iv * (1.5 - 0.5 * xv * iv * iv)
            for s in range(_NSL):
                dsl = pl.ds(s * _L, _L)
                v = wbuf[t, dsl]
                wbuf[t, dsl] = (v - mean) * iv * gamma_v[dsl] + beta_v[dsl]

    def step(k, wx, semx, osemx, wy, semy, osemy, first, last):
        c0, base8 = chunk_scalars(k)
        build_spatial_idx(c0)
        issue_side(base8)
        wait_word(wx, semx)
        if not last:
            if not first:
                wait_out(wy, osemy)
            issue_word(c0 + _T, wy, semy)
        wait_side()
        compute(c0, base8, wx)
        out_copy(c0, wx, osemx)

    issue_word(0, w_a, sem_wa)
    step(0, w_a, sem_wa, osem_a, w_b, sem_wb, osem_b, True, False)

    def pipe_body(j, _):
        step(1 + 2 * j, w_b, sem_wb, osem_b, w_a, sem_wa, osem_a, False, False)
        step(2 + 2 * j, w_a, sem_wa, osem_a, w_b, sem_wb, osem_b, False, False)
        return 0

    lax.fori_loop(0, (_NCHUNK - 2) // 2, pipe_body, 0)

    step(_NCHUNK - 1, w_b, sem_wb, osem_b, w_a, sem_wa, osem_a, False, True)
    wait_out(w_a, osem_a)
    wait_out(w_b, osem_b)


@jax.jit
def _run(input_ids, bbT, word2, pos_pad, xy_cat, hw_cat, ln_gamma, ln_beta):
    k = functools.partial(
        pl.kernel,
        out_type=jax.ShapeDtypeStruct((_B, _S, _HID), jnp.float32),
        mesh=plsc.VectorSubcoreMesh(core_axis_name="c", subcore_axis_name="s"),
        compiler_params=pltpu.CompilerParams(needs_layout_passes=False),
        scratch_types=[
            pltpu.VMEM_SHARED((2 * _MAX_2D, 128), jnp.float32),
            pltpu.VMEM((_S,), jnp.int32),
            pltpu.VMEM((4, _S), jnp.int32),
            pltpu.VMEM((_S + _L,), jnp.int32),
            pltpu.VMEM((_HID,), jnp.float32),
            pltpu.VMEM((_HID,), jnp.float32),
            pltpu.VMEM((128,), jnp.int32),
            pltpu.VMEM((64,), jnp.int32),
            pltpu.VMEM((_PW, _HID), jnp.float32),
            pltpu.VMEM((_T, _HID), jnp.float32),
            pltpu.VMEM((_T, _HID), jnp.float32),
            pltpu.VMEM((128, 128), jnp.float32),
            pltpu.VMEM((64, 128), jnp.float32),
            pltpu.SemaphoreType.DMA,
            pltpu.SemaphoreType.DMA,
            pltpu.SemaphoreType.DMA,
            pltpu.SemaphoreType.DMA,
            pltpu.SemaphoreType.DMA,
            pltpu.SemaphoreType.DMA,
        ],
    )(_sc_body)
    return k(input_ids, bbT, word2, pos_pad, xy_cat, hw_cat, ln_gamma, ln_beta)


def kernel(input_ids, bbox, word_emb, token_type_emb, pos_emb, x_emb, y_emb,
           h_emb, w_emb, ln_gamma, ln_beta):
    pos_plus = pos_emb + token_type_emb[0][None, :]
    word2 = word_emb.at[_PAD].add(pos_plus[_PAD])
    pos_pad = jnp.pad(pos_plus, ((0, _POSPAD - pos_plus.shape[0]), (0, 0)))
    xy_cat = jnp.concatenate([x_emb, y_emb], axis=0)
    hw_cat = jnp.concatenate([h_emb, w_emb], axis=0)
    bbT = jnp.transpose(bbox, (2, 0, 1))
    return _run(input_ids, bbT, word2, pos_pad, xy_cat, hw_cat,
                ln_gamma, ln_beta)


# speedup vs baseline: 1.0908x; 1.0908x over previous
import functools

import jax
import jax.numpy as jnp
from jax import lax
from jax.experimental import pallas as pl
from jax.experimental.pallas import tpu as pltpu
from jax.experimental.pallas import tpu_sc as plsc

_HID = 768
_MAX_2D = 1024
_PAD = 1
_EPS = 1e-5
_B, _S = 32, 512
_T = 16
_NCHUNK = _S // _T
_L = 16
_NSL = _HID // _L
_SEG = 128 // _L
_PW = 24
_POSPAD = 640


def _sc_body(ids_hbm, bbT_hbm, word_hbm, pos_hbm, xyc_hbm, hwc_hbm,
             gamma_hbm, beta_hbm, out_hbm,
             ids_v, bb_v, rm_v, gamma_v, beta_v,
             set_a, set_b, sem_a, sem_b, osem_a, osem_b):
    sid = lax.axis_index("s")
    b = sid * 2 + lax.axis_index("c")

    pltpu.sync_copy(ids_hbm.at[b], ids_v)
    for c in range(4):
        pltpu.sync_copy(bbT_hbm.at[c, b], bb_v.at[c])
    pltpu.sync_copy(gamma_hbm, gamma_v)
    pltpu.sync_copy(beta_hbm, beta_v)

    def idx_body(i, carry):
        sl = pl.ds(i * _L, _L)
        ids = ids_v[sl]
        m = jnp.where(ids == _PAD, 0, 1).astype(jnp.int32)
        cs = lax.cumsum(m, axis=0) + carry
        rm_v[sl] = ((cs + 1) << 1) | m
        return carry + jnp.sum(m)

    lax.fori_loop(0, _S // _L, idx_body, jnp.int32(0))

    def issue(c0, bufs, sem):
        w_v, pos_rows, g1_v, g2_v, ig1c, ig2c, irc = bufs
        sl16 = pl.ds(c0, _L)
        irc[pl.ds(0, _L)] = rm_v[sl16] >> 1
        b0 = bb_v[0, sl16]
        b1 = bb_v[1, sl16]
        b2 = bb_v[2, sl16]
        b3 = bb_v[3, sl16]
        ig1c[pl.ds(0, _L)] = b0
        ig1c[pl.ds(16, _L)] = b1 + _MAX_2D
        ig1c[pl.ds(32, _L)] = b2
        ig1c[pl.ds(48, _L)] = b3 + _MAX_2D
        ig2c[pl.ds(0, _L)] = jnp.clip(b3 - b1, 0, _MAX_2D - 1)
        ig2c[pl.ds(16, _L)] = jnp.clip(b2 - b0, 0, _MAX_2D - 1) + _MAX_2D
        pltpu.async_copy(word_hbm.at[ids_v.at[sl16]], w_v, sem)
        pltpu.async_copy(pos_hbm.at[irc], pos_rows, sem)
        pltpu.async_copy(xyc_hbm.at[ig1c], g1_v, sem)
        pltpu.async_copy(hwc_hbm.at[ig2c], g2_v, sem)

    def wait_gathers(bufs, sem):
        w_v, pos_rows, g1_v, g2_v, _, _, _ = bufs
        pltpu.make_async_copy(word_hbm.at[pl.ds(0, _T)], w_v, sem).wait()
        pltpu.make_async_copy(pos_hbm.at[pl.ds(0, _T)], pos_rows, sem).wait()
        pltpu.make_async_copy(xyc_hbm.at[pl.ds(0, 64)], g1_v, sem).wait()
        pltpu.make_async_copy(hwc_hbm.at[pl.ds(0, 32)], g2_v, sem).wait()

    def out_copy(c0, bufs, osem):
        pltpu.async_copy(bufs[0], out_hbm.at[b, pl.ds(c0, _T)], osem)

    def wait_out(bufs, osem):
        pltpu.make_async_copy(bufs[0], out_hbm.at[b, pl.ds(0, _T)], osem).wait()

    def compute(c0, bufs):
        w_v, pos_rows, g1_v, g2_v, _, _, _ = bufs

        @plsc.parallel_loop(0, _T, 1, unroll=2)
        def token_body(t):
            rm = rm_v[pl.ds(c0 + t, _L)][0]
            mt = (rm & 1).astype(jnp.float32)
            vs = [jnp.zeros((_L,), jnp.float32) for _ in range(4)]
            vq = [jnp.zeros((_L,), jnp.float32) for _ in range(4)]
            for s in range(_NSL):
                dsl = pl.ds(s * _L, _L)
                seg = s // _SEG
                sub = pl.ds((s % _SEG) * _L, _L)
                v = w_v[t, dsl] + pos_rows[t, dsl] * mt
                if seg < 4:
                    v = v + g1_v[seg * _T + t, sub]
                else:
                    v = v + g2_v[(seg - 4) * _T + t, sub]
                w_v[t, dsl] = v
                a = s % 4
                vs[a] = vs[a] + v
                vq[a] = vq[a] + v * v
            vsum = (vs[0] + vs[1]) + (vs[2] + vs[3])
            vsq = (vq[0] + vq[1]) + (vq[2] + vq[3])
            mean = jnp.sum(vsum) * (1.0 / _HID)
            var = jnp.sum(vsq) * (1.0 / _HID) - mean * mean + _EPS
            xv = jnp.full((_L,), var, dtype=jnp.float32)
            iv = lax.bitcast_convert_type(
                jnp.int32(0x5F3759DF)
                - (lax.bitcast_convert_type(xv, jnp.int32) >> 1),
                jnp.float32)
            for _ in range(3):
                iv = iv * (1.5 - 0.5 * xv * iv * iv)
            for s in range(_NSL):
                dsl = pl.ds(s * _L, _L)
                v = w_v[t, dsl]
                w_v[t, dsl] = (v - mean) * iv * gamma_v[dsl] + beta_v[dsl]

    issue(0, set_a, sem_a)
    issue(_T, set_b, sem_b)
    wait_gathers(set_a, sem_a)
    compute(0, set_a)
    out_copy(0, set_a, osem_a)

    def pipe_body(j, _):
        c1 = (1 + 2 * j) * _T
        wait_out(set_a, osem_a)
        issue(c1 + _T, set_a, sem_a)
        wait_gathers(set_b, sem_b)
        compute(c1, set_b)
        out_copy(c1, set_b, osem_b)

        c2 = c1 + _T
        wait_out(set_b, osem_b)
        issue(c2 + _T, set_b, sem_b)
        wait_gathers(set_a, sem_a)
        compute(c2, set_a)
        out_copy(c2, set_a, osem_a)
        return 0

    lax.fori_loop(0, (_NCHUNK - 2) // 2, pipe_body, 0)

    c_last = (_NCHUNK - 1) * _T
    wait_gathers(set_b, sem_b)
    compute(c_last, set_b)
    out_copy(c_last, set_b, osem_b)
    wait_out(set_a, osem_a)
    wait_out(set_b, osem_b)


def _buf_set():
    return (
        pltpu.VMEM((_T, _HID), jnp.float32),
        pltpu.VMEM((_T, _HID), jnp.float32),
        pltpu.VMEM((64, 128), jnp.float32),
        pltpu.VMEM((32, 128), jnp.float32),
        pltpu.VMEM((64,), jnp.int32),
        pltpu.VMEM((32,), jnp.int32),
        pltpu.VMEM((_T,), jnp.int32),
    )


@jax.jit
def _run(input_ids, bbT, word2, pos_pad, xy_cat, hw_cat, ln_gamma, ln_beta):
    k = functools.partial(
        pl.kernel,
        out_type=jax.ShapeDtypeStruct((_B, _S, _HID), jnp.float32),
        mesh=plsc.VectorSubcoreMesh(core_axis_name="c", subcore_axis_name="s"),
        compiler_params=pltpu.CompilerParams(needs_layout_passes=False),
        scratch_types=[
            pltpu.VMEM((_S,), jnp.int32),
            pltpu.VMEM((4, _S), jnp.int32),
            pltpu.VMEM((_S + _L,), jnp.int32),
            pltpu.VMEM((_HID,), jnp.float32),
            pltpu.VMEM((_HID,), jnp.float32),
            _buf_set(),
            _buf_set(),
            pltpu.SemaphoreType.DMA,
            pltpu.SemaphoreType.DMA,
            pltpu.SemaphoreType.DMA,
            pltpu.SemaphoreType.DMA,
        ],
    )(_sc_body)
    return k(input_ids, bbT, word2, pos_pad, xy_cat, hw_cat, ln_gamma, ln_beta)


def kernel(input_ids, bbox, word_emb, token_type_emb, pos_emb, x_emb, y_emb,
           h_emb, w_emb, ln_gamma, ln_beta):
    pos_plus = pos_emb + token_type_emb[0][None, :]
    word2 = word_emb.at[_PAD].add(pos_plus[_PAD])
    pos_pad = pos_plus
    xy_cat = jnp.concatenate([x_emb, y_emb], axis=0)
    hw_cat = jnp.concatenate([h_emb, w_emb], axis=0)
    bbT = jnp.transpose(bbox, (2, 0, 1))
    return _run(input_ids, bbT, word2, pos_pad, xy_cat, hw_cat,
                ln_gamma, ln_beta)


# speedup vs baseline: 1.3028x; 1.1944x over previous
import functools

import jax
import jax.numpy as jnp
from jax import lax
from jax.experimental import pallas as pl
from jax.experimental.pallas import tpu as pltpu
from jax.experimental.pallas import tpu_sc as plsc

_HID = 768
_MAX_2D = 1024
_PAD = 1
_EPS = 1e-5
_B, _S = 32, 512
_T = 16
_NCHUNK = _S // _T
_L = 16
_NSL = _HID // _L
_SEG = 128 // _L


def _sc_body(ids_hbm, bbT_hbm, word_hbm, pos_hbm, x_hbm, y_hbm, h_hbm, w_hbm,
             gamma_hbm, beta_hbm, out_hbm,
             ids_v, bb_v, pos_idx_v, h_idx_v, w_idx_v, gamma_v, beta_v,
             bufs_a, bufs_b, sem_a, sem_b, out_sem_a, out_sem_b):
    wid = lax.axis_index("s") * 2 + lax.axis_index("c")
    b = wid

    pltpu.sync_copy(ids_hbm.at[b], ids_v)
    for c in range(4):
        pltpu.sync_copy(bbT_hbm.at[c, b], bb_v.at[c])
    pltpu.sync_copy(gamma_hbm, gamma_v)
    pltpu.sync_copy(beta_hbm, beta_v)

    def idx_body(i, carry):
        sl = pl.ds(i * _L, _L)
        ids = ids_v[sl]
        m = jnp.where(ids == _PAD, 0, 1).astype(jnp.int32)
        cs = lax.cumsum(m, axis=0) + carry
        pos_idx_v[sl] = cs * m + _PAD
        b0 = bb_v[0, sl]
        b1 = bb_v[1, sl]
        b2 = bb_v[2, sl]
        b3 = bb_v[3, sl]
        h_idx_v[sl] = jnp.clip(b3 - b1, 0, _MAX_2D - 1)
        w_idx_v[sl] = jnp.clip(b2 - b0, 0, _MAX_2D - 1)
        return carry + jnp.sum(m)

    lax.fori_loop(0, _S // _L, idx_body, jnp.int32(0))

    def issue(c0, bufs, sem):
        sl = pl.ds(c0, _T)
        word_buf, pos_buf, sx0, sy1, sx2, sy3, sh, sw = bufs
        pltpu.async_copy(word_hbm.at[ids_v.at[sl]], word_buf, sem)
        pltpu.async_copy(pos_hbm.at[pos_idx_v.at[sl]], pos_buf, sem)
        pltpu.async_copy(x_hbm.at[bb_v.at[0, sl]], sx0, sem)
        pltpu.async_copy(y_hbm.at[bb_v.at[1, sl]], sy1, sem)
        pltpu.async_copy(x_hbm.at[bb_v.at[2, sl]], sx2, sem)
        pltpu.async_copy(y_hbm.at[bb_v.at[3, sl]], sy3, sem)
        pltpu.async_copy(h_hbm.at[h_idx_v.at[sl]], sh, sem)
        pltpu.async_copy(w_hbm.at[w_idx_v.at[sl]], sw, sem)

    def wait_gathers(bufs, sem):
        word_buf, pos_buf, sx0, sy1, sx2, sy3, sh, sw = bufs
        dummy = pl.ds(0, _T)
        pltpu.make_async_copy(word_hbm.at[dummy], word_buf, sem).wait()
        pltpu.make_async_copy(pos_hbm.at[dummy], pos_buf, sem).wait()
        pltpu.make_async_copy(x_hbm.at[dummy], sx0, sem).wait()
        pltpu.make_async_copy(y_hbm.at[dummy], sy1, sem).wait()
        pltpu.make_async_copy(x_hbm.at[dummy], sx2, sem).wait()
        pltpu.make_async_copy(y_hbm.at[dummy], sy3, sem).wait()
        pltpu.make_async_copy(h_hbm.at[dummy], sh, sem).wait()
        pltpu.make_async_copy(w_hbm.at[dummy], sw, sem).wait()

    def compute(bufs):
        word_buf, pos_buf, sx0, sy1, sx2, sy3, sh, sw = bufs
        spat = (sx0, sy1, sx2, sy3, sh, sw)

        def token_body(t, _):
            vsum = jnp.zeros((_L,), jnp.float32)
            vsq = jnp.zeros((_L,), jnp.float32)
            for s in range(_NSL):
                dsl = pl.ds(s * _L, _L)
                v = word_buf[t, dsl] + pos_buf[t, dsl]
                v = v + spat[s // _SEG][t, pl.ds((s % _SEG) * _L, _L)]
                word_buf[t, dsl] = v
                vsum = vsum + v
                vsq = vsq + v * v
            mean = jnp.sum(vsum) * (1.0 / _HID)
            var = jnp.sum(vsq) * (1.0 / _HID) - mean * mean + _EPS
            xv = jnp.full((_L,), var, dtype=jnp.float32)
            iv = lax.bitcast_convert_type(
                jnp.int32(0x5F3759DF)
                - (lax.bitcast_convert_type(xv, jnp.int32) >> 1),
                jnp.float32)
            for _ in range(3):
                iv = iv * (1.5 - 0.5 * xv * iv * iv)
            for s in range(_NSL):
                dsl = pl.ds(s * _L, _L)
                v = word_buf[t, dsl]
                word_buf[t, dsl] = (v - mean) * iv * gamma_v[dsl] + beta_v[dsl]
            return 0

        lax.fori_loop(0, _T, token_body, 0)

    def out_copy(c0, bufs, osem):
        pltpu.async_copy(bufs[0], out_hbm.at[b, pl.ds(c0, _T)], osem)

    def wait_out(bufs, osem):
        pltpu.make_async_copy(bufs[0], out_hbm.at[b, pl.ds(0, _T)], osem).wait()

    issue(0, bufs_a, sem_a)
    issue(_T, bufs_b, sem_b)
    wait_gathers(bufs_a, sem_a)
    compute(bufs_a)
    out_copy(0, bufs_a, out_sem_a)

    def pipe_body(j, _):
        c1 = (1 + 2 * j) * _T
        wait_out(bufs_a, out_sem_a)
        issue(c1 + _T, bufs_a, sem_a)
        wait_gathers(bufs_b, sem_b)
        compute(bufs_b)
        out_copy(c1, bufs_b, out_sem_b)

        c2 = c1 + _T
        wait_out(bufs_b, out_sem_b)
        issue(c2 + _T, bufs_b, sem_b)
        wait_gathers(bufs_a, sem_a)
        compute(bufs_a)
        out_copy(c2, bufs_a, out_sem_a)
        return 0

    lax.fori_loop(0, (_NCHUNK - 2) // 2, pipe_body, 0)

    wait_gathers(bufs_b, sem_b)
    compute(bufs_b)
    out_copy((_NCHUNK - 1) * _T, bufs_b, out_sem_b)
    wait_out(bufs_a, out_sem_a)
    wait_out(bufs_b, out_sem_b)


def _buf_set():
    return (
        pltpu.VMEM((_T, _HID), jnp.float32),
        pltpu.VMEM((_T, _HID), jnp.float32),
        pltpu.VMEM((_T, 128), jnp.float32),
        pltpu.VMEM((_T, 128), jnp.float32),
        pltpu.VMEM((_T, 128), jnp.float32),
        pltpu.VMEM((_T, 128), jnp.float32),
        pltpu.VMEM((_T, 128), jnp.float32),
        pltpu.VMEM((_T, 128), jnp.float32),
    )


@jax.jit
def _run(input_ids, bbT, word_emb, pos_plus, x_emb, y_emb, h_emb, w_emb,
         ln_gamma, ln_beta):
    k = functools.partial(
        pl.kernel,
        out_type=jax.ShapeDtypeStruct((_B, _S, _HID), jnp.float32),
        mesh=plsc.VectorSubcoreMesh(core_axis_name="c", subcore_axis_name="s"),
        compiler_params=pltpu.CompilerParams(needs_layout_passes=False),
        scratch_types=[
            pltpu.VMEM((_S,), jnp.int32),
            pltpu.VMEM((4, _S), jnp.int32),
            pltpu.VMEM((_S,), jnp.int32),
            pltpu.VMEM((_S,), jnp.int32),
            pltpu.VMEM((_S,), jnp.int32),
            pltpu.VMEM((_HID,), jnp.float32),
            pltpu.VMEM((_HID,), jnp.float32),
            _buf_set(),
            _buf_set(),
            pltpu.SemaphoreType.DMA,
            pltpu.SemaphoreType.DMA,
            pltpu.SemaphoreType.DMA,
            pltpu.SemaphoreType.DMA,
        ],
    )(_sc_body)
    return k(input_ids, bbT, word_emb, pos_plus, x_emb, y_emb, h_emb, w_emb,
             ln_gamma, ln_beta)


def kernel(input_ids, bbox, word_emb, token_type_emb, pos_emb, x_emb, y_emb,
           h_emb, w_emb, ln_gamma, ln_beta):
    pos_plus = pos_emb + token_type_emb[0][None, :]
    bbT = jnp.transpose(bbox, (2, 0, 1))
    return _run(input_ids, bbT, word_emb, pos_plus, x_emb, y_emb, h_emb,
                w_emb, ln_gamma, ln_beta)
